# baseline (device time: 121991 ns/iter reference)
import functools

import jax
import jax.numpy as jnp
from jax import lax
from jax.experimental import pallas as pl
from jax.experimental.pallas import tpu as pltpu


def _allreduce_y(part):
    n_tok, d = part.shape

    def body(part_ref, out_ref, recv_ref, send_sem, recv_sem):
        my_x = lax.axis_index("x")
        my_y = lax.axis_index("y")
        nbr = (my_x, 1 - my_y)

        barrier = pltpu.get_barrier_semaphore()
        pl.semaphore_signal(
            barrier, inc=1, device_id=nbr, device_id_type=pl.DeviceIdType.MESH
        )
        pl.semaphore_wait(barrier, 1)

        rdma = pltpu.make_async_remote_copy(
            src_ref=part_ref,
            dst_ref=recv_ref,
            send_sem=send_sem,
            recv_sem=recv_sem,
            device_id=nbr,
            device_id_type=pl.DeviceIdType.MESH,
        )
        rdma.start()
        rdma.wait()

        out_ref[...] = part_ref[...].astype(jnp.float32) + recv_ref[...].astype(
            jnp.float32
        )

        @functools.partial(pl.run_scoped, sem2=pltpu.SemaphoreType.REGULAR)
        def _(sem2):
            pl.semaphore_signal(
                sem2, inc=1, device_id=nbr, device_id_type=pl.DeviceIdType.MESH
            )
            pl.semaphore_wait(sem2, 1)

    return pl.pallas_call(
        body,
        out_shape=jax.ShapeDtypeStruct((n_tok, d), jnp.float32),
        in_specs=[pl.BlockSpec(memory_space=pltpu.VMEM)],
        out_specs=pl.BlockSpec(memory_space=pltpu.VMEM),
        scratch_shapes=[
            pltpu.VMEM((n_tok, d), jnp.bfloat16),
            pltpu.SemaphoreType.DMA,
            pltpu.SemaphoreType.DMA,
        ],
        compiler_params=pltpu.CompilerParams(collective_id=0),
    )(part)


def kernel(ids, E):
    v_per, _ = E.shape
    my_y = lax.axis_index("y")
    base = my_y * v_per
    local = ids - base
    mask = (local >= 0) & (local < v_per)
    idx = jnp.where(mask, local, 0)
    part = jnp.take(E, idx, axis=0, mode="clip")
    part = jnp.where(mask[:, None], part, 0.0).astype(jnp.bfloat16)
    return _allreduce_y(part)


# device time: 61680 ns/iter; 1.9778x vs baseline; 1.9778x over previous
import jax
import jax.numpy as jnp
from jax import lax
from jax.experimental import pallas as pl
from jax.experimental.pallas import tpu as pltpu

N_CHUNK = 8


def _lookup_allreduce(idx, maskv, E, n_tok_half, d):
    rows = n_tok_half // N_CHUNK

    def body(idx_ref, mask_ref, e_ref, out_ref, gbuf, sbuf, yrecv, xbuf, xrecv,
             g_sems, y_send, y_recv, x_send, x_recv):
        my_x = lax.axis_index("x")
        my_y = lax.axis_index("y")
        ynbr = (my_x, 1 - my_y)
        xnbr = (1 - my_x, my_y)

        for c in range(N_CHUNK):
            def issue(t, _, c=c):
                tt = c * rows + t
                pltpu.make_async_copy(
                    e_ref.at[pl.ds(idx_ref[tt], 1), :],
                    gbuf.at[pl.ds(tt, 1), :],
                    g_sems.at[c],
                ).start()
                return 0
            lax.fori_loop(0, rows, issue, 0)

        barrier = pltpu.get_barrier_semaphore()
        for nbr in (ynbr, xnbr):
            pl.semaphore_signal(
                barrier, inc=1, device_id=nbr,
                device_id_type=pl.DeviceIdType.MESH,
            )
        pl.semaphore_wait(barrier, 2)

        y_rdma = []
        x_rdma = []
        for c in range(N_CHUNK):
            sl = pl.ds(c * rows, rows)
            y_rdma.append(pltpu.make_async_remote_copy(
                src_ref=sbuf.at[sl, :], dst_ref=yrecv.at[sl, :],
                send_sem=y_send.at[c], recv_sem=y_recv.at[c],
                device_id=ynbr, device_id_type=pl.DeviceIdType.MESH,
            ))
            x_rdma.append(pltpu.make_async_remote_copy(
                src_ref=xbuf.at[sl, :], dst_ref=xrecv.at[sl, :],
                send_sem=x_send.at[c], recv_sem=x_recv.at[c],
                device_id=xnbr, device_id_type=pl.DeviceIdType.MESH,
            ))

        for c in range(N_CHUNK):
            def drain(t, _, c=c):
                pltpu.make_async_copy(
                    e_ref.at[pl.ds(0, 1), :], gbuf.at[pl.ds(0, 1), :],
                    g_sems.at[c],
                ).wait()
                return 0
            lax.fori_loop(0, rows, drain, 0)
            sl = pl.ds(c * rows, rows)
            sbuf[sl, :] = (gbuf[sl, :] * mask_ref[sl, :]).astype(jnp.bfloat16)
            y_rdma[c].start()

        for c in range(N_CHUNK):
            y_rdma[c].wait_recv()
            sl = pl.ds(c * rows, rows)
            res = sbuf[sl, :] + yrecv[sl, :]
            out_ref[pl.ds(my_x * n_tok_half + c * rows, rows), :] = res.astype(
                jnp.float32
            )
            xbuf[sl, :] = res
            x_rdma[c].start()

        for c in range(N_CHUNK):
            x_rdma[c].wait_recv()
            sl = pl.ds(c * rows, rows)
            out_ref[pl.ds((1 - my_x) * n_tok_half + c * rows, rows), :] = (
                xrecv[sl, :].astype(jnp.float32)
            )

        for c in range(N_CHUNK):
            y_rdma[c].wait_send()
            x_rdma[c].wait_send()

    return pl.pallas_call(
        body,
        out_shape=jax.ShapeDtypeStruct((2 * n_tok_half, d), jnp.float32),
        in_specs=[
            pl.BlockSpec(memory_space=pltpu.SMEM),
            pl.BlockSpec(memory_space=pltpu.VMEM),
            pl.BlockSpec(memory_space=pl.ANY),
        ],
        out_specs=pl.BlockSpec(memory_space=pltpu.VMEM),
        scratch_shapes=[
            pltpu.VMEM((n_tok_half, d), jnp.float32),
            pltpu.VMEM((n_tok_half, d), jnp.bfloat16),
            pltpu.VMEM((n_tok_half, d), jnp.bfloat16),
            pltpu.VMEM((n_tok_half, d), jnp.bfloat16),
            pltpu.VMEM((n_tok_half, d), jnp.bfloat16),
            pltpu.SemaphoreType.DMA((N_CHUNK,)),
            pltpu.SemaphoreType.DMA((N_CHUNK,)),
            pltpu.SemaphoreType.DMA((N_CHUNK,)),
            pltpu.SemaphoreType.DMA((N_CHUNK,)),
            pltpu.SemaphoreType.DMA((N_CHUNK,)),
        ],
        compiler_params=pltpu.CompilerParams(collective_id=0),
    )(idx, maskv, E)


def kernel(ids, E):
    v_per, d = E.shape
    n_tok = ids.shape[0]
    half = n_tok // 2
    my_x = lax.axis_index("x")
    my_y = lax.axis_index("y")
    ids_half = lax.dynamic_slice(ids, (my_x * half,), (half,))
    local = ids_half - my_y * v_per
    mask = (local >= 0) & (local < v_per)
    idx = jnp.clip(local, 0, v_per - 1)
    maskv = mask.astype(jnp.float32)[:, None]
    return _lookup_allreduce(idx, maskv, E, half, d)
